# baseline (speedup 1.0000x reference)
import functools

import jax
import jax.numpy as jnp
from jax import lax
from jax.experimental import pallas as pl
from jax.experimental.pallas import tpu as pltpu
from jax.experimental.pallas import tpu_sc as plsc

_MAX_REL = 128
_EMB = 16
_VOCAB = 2 * _MAX_REL + 1
_LEN = 2048
_SLEN = 2 * _LEN
_NC = 2
_NS = 16
_NW = _NC * _NS
_ROWS_PER_W = _LEN // _NW


@functools.partial(
    pl.kernel,
    out_type=jax.ShapeDtypeStruct((_LEN, 2, 16, 8, 128), jnp.float32),
    mesh=plsc.VectorSubcoreMesh(core_axis_name="c", subcore_axis_name="s"),
    scratch_types=[
        pltpu.VMEM((_VOCAB, _EMB), jnp.float32),
        pltpu.VMEM((2, 8, _SLEN), jnp.float32),
        pltpu.SemaphoreType.DMA,
    ],
    compiler_params=pltpu.CompilerParams(use_tc_tiling_on_sc=False, needs_layout_passes=False),
)
def _rpe_sc(table_hbm, out_hbm, table_v, sarr_v, sem):
    wid = lax.axis_index("s") * _NC + lax.axis_index("c")
    r = wid % 8
    q = wid // 8
    phi = (7 - r) % 8

    pltpu.sync_copy(table_hbm, table_v)

    lane = lax.iota(jnp.int32, 16)

    off_min = 1536 - 512 * q

    def build_e(e, carry):
        ef = jnp.full((16,), e, jnp.int32)
        d0 = e // 8
        d1 = e % 8

        def build_chunk(c, carry2):
            m0 = off_min + 16 * c
            row = jnp.clip(2047 - phi - m0 - lane, -_MAX_REL, _MAX_REL) + _MAX_REL
            v = plsc.load_gather(table_v, [row, ef])
            sarr_v[d0, d1, pl.ds(m0, 16)] = v
            return carry2

        lax.fori_loop(0, 160, build_chunk, 0, unroll=4)
        return carry

    lax.fori_loop(0, _EMB, build_e, 0)

    def copy_plane(k, carry):
        i = r + 512 * q + 8 * k
        off = pl.multiple_of((2047 - i) - phi, 8)
        descs = [
            pltpu.async_copy(
                sarr_v.at[:, :, pl.ds(pl.multiple_of(off + 128 * t, 8), 128)],
                out_hbm.at[i, :, t],
                sem,
            )
            for t in range(16)
        ]

        @pl.when(k > 0)
        def _retire_previous_round():
            for d in descs:
                d.wait()

        return carry

    lax.fori_loop(0, _ROWS_PER_W, copy_plane, 0)

    i_last = r + 512 * q + 8 * (_ROWS_PER_W - 1)
    off_last = pl.multiple_of((2047 - i_last) - phi, 8)
    for t in range(16):
        pltpu.make_async_copy(
            sarr_v.at[:, :, pl.ds(pl.multiple_of(off_last + 128 * t, 8), 128)],
            out_hbm.at[i_last, :, t],
            sem,
        ).wait()


def kernel(length, table):
    del length
    out = _rpe_sc(table)
    out = out.transpose(0, 2, 4, 1, 3)
    return out.reshape(_LEN, _LEN, _EMB)


# speedup vs baseline: 1.0066x; 1.0066x over previous
import functools

import jax
import jax.numpy as jnp
from jax import lax
from jax.experimental import pallas as pl
from jax.experimental.pallas import tpu as pltpu
from jax.experimental.pallas import tpu_sc as plsc

_MAX_REL = 128
_EMB = 16
_VOCAB = 2 * _MAX_REL + 1
_LEN = 2048
_SLEN = 2 * _LEN
_NC = 2
_NS = 16
_NW = _NC * _NS
_ROWS_PER_W = _LEN // _NW


@functools.partial(
    pl.kernel,
    out_type=jax.ShapeDtypeStruct((_LEN, 2, 16, 8, 128), jnp.float32),
    mesh=plsc.VectorSubcoreMesh(core_axis_name="c", subcore_axis_name="s"),
    scratch_types=[
        pltpu.VMEM((_VOCAB, _EMB), jnp.float32),
        pltpu.VMEM((2, 8, _SLEN), jnp.float32),
        pltpu.SemaphoreType.DMA,
    ],
    compiler_params=pltpu.CompilerParams(use_tc_tiling_on_sc=False, needs_layout_passes=False),
)
def _rpe_sc(table_hbm, out_hbm, table_v, sarr_v, sem):
    wid = lax.axis_index("s") * _NC + lax.axis_index("c")
    r = wid % 8
    q = wid // 8
    phi = (7 - r) % 8

    pltpu.sync_copy(table_hbm, table_v)

    lane = lax.iota(jnp.int32, 16)

    off_min = 1536 - 512 * q

    def build_e(e, carry):
        ef = jnp.full((16,), e, jnp.int32)
        d0 = e // 8
        d1 = e % 8

        def build_chunk(c, carry2):
            m0 = off_min + 16 * c
            row = jnp.clip(2047 - phi - m0 - lane, -_MAX_REL, _MAX_REL) + _MAX_REL
            v = plsc.load_gather(table_v, [row, ef])
            sarr_v[d0, d1, pl.ds(m0, 16)] = v
            return carry2

        lax.fori_loop(0, 160, build_chunk, 0, unroll=4)
        return carry

    lax.fori_loop(0, _EMB, build_e, 0)

    def copy_plane(k, carry):
        i = r + 512 * q + 8 * k
        off = pl.multiple_of((2047 - i) - phi, 8)
        descs = [
            pltpu.async_copy(
                sarr_v.at[:, :, pl.ds(pl.multiple_of(off + 128 * t, 8), 128)],
                out_hbm.at[i, :, t],
                sem,
            )
            for t in range(16)
        ]
        for d in descs:
            d.wait()
        return carry

    lax.fori_loop(0, _ROWS_PER_W, copy_plane, 0)


def kernel(length, table):
    del length
    out = _rpe_sc(table)
    out = out.transpose(0, 2, 4, 1, 3)
    return out.reshape(_LEN, _LEN, _EMB)
